# Initial kernel scaffold; baseline (speedup 1.0000x reference)
import jax
import jax.numpy as jnp
from jax.experimental import pallas as pl


def kernel(x, edge_index, edge_type, W1, b1, W1s, b1s, W2, b2, W2s, b2s):
    raise NotImplementedError("write your pallas kernel here")


# speedup vs baseline: 3.9264x; 3.9264x over previous
import jax
import jax.numpy as jnp
from jax import lax
from jax.experimental import pallas as pl
from jax.experimental.pallas import tpu as pltpu
from jax.experimental.pallas import tpu_sc as plsc

N = 10000
E = 320000
R = 16
D = 128
EPS = 1e-10

NC = 2
NS = 16
NW = NC * NS
EPW = E // NW
WIN = 80
NWIN = EPW // WIN
SEG = N * R
SEG_PER_SUB = SEG // NS
ROW_MAIN = 624
ROW_TAIL = N - NS * ROW_MAIN

_MESH = plsc.VectorSubcoreMesh(core_axis_name="c", subcore_axis_name="s")

_NBLK = 25
_BROW = N // _NBLK


CH = 5
NCHUNK = NWIN // CH
ZROWS = 104
DZ = 2000


def _den_body(dst_hbm, rel_hbm, den2_hbm, dstc, relc, segv, onesv, zbuf, den_sh):
    c = lax.axis_index("c")
    s = lax.axis_index("s")
    wid = c * NS + s

    @pl.loop(0, DZ // 16)
    def _(i):
        zbuf[pl.ds(i * 16, 16)] = jnp.zeros((16,), jnp.float32)

    for q in range(SEG_PER_SUB // DZ):
        pltpu.sync_copy(zbuf, den_sh.at[pl.ds(s * SEG_PER_SUB + q * DZ, DZ)])

    for k in range(WIN // 16):
        onesv[pl.ds(k * 16, 16)] = jnp.ones((16,), jnp.float32)

    plsc.subcore_barrier()

    @pl.loop(0, NCHUNK)
    def _(t0):
        pltpu.sync_copy(dst_hbm.at[wid, pl.ds(t0 * CH, CH)], dstc)
        pltpu.sync_copy(rel_hbm.at[wid, pl.ds(t0 * CH, CH)], relc)
        for tt in range(CH):
            for k in range(WIN // 16):
                d = dstc[tt, 0, pl.ds(k * 16, 16)]
                r = relc[tt, 0, pl.ds(k * 16, 16)]
                segv[0, pl.ds(k * 16, 16)] = d * R + r
            pltpu.sync_copy(onesv, den_sh.at[segv.at[0]], add=True)

    plsc.subcore_barrier()
    for q in range(SEG_PER_SUB // DZ):
        pltpu.sync_copy(den_sh.at[pl.ds(s * SEG_PER_SUB + q * DZ, DZ)], zbuf)
        pltpu.sync_copy(zbuf,
                        den2_hbm.at[pl.ds(c * SEG + s * SEG_PER_SUB + q * DZ,
                                          DZ)])


def _den_call(dst4d, rel4d):
    return pl.kernel(
        _den_body,
        out_type=jax.ShapeDtypeStruct((NC * SEG,), jnp.float32),
        mesh=_MESH,
        scratch_types=[
            pltpu.VMEM((CH, 1, WIN), jnp.int32),
            pltpu.VMEM((CH, 1, WIN), jnp.int32),
            pltpu.VMEM((1, WIN), jnp.int32),
            pltpu.VMEM((WIN,), jnp.float32),
            pltpu.VMEM((DZ,), jnp.float32),
            pltpu.VMEM_SHARED((SEG,), jnp.float32),
        ],
    )(dst4d, rel4d)


def _agg_body(src_hbm, dst_hbm, rel_hbm, inv_hbm, y_hbm, out_hbm,
              srcc, relc, dstc, segv, srv, invv, rows, zbuf, acc_sh):
    c = lax.axis_index("c")
    s = lax.axis_index("s")
    wid = c * NS + s

    @pl.loop(0, ZROWS)
    def _(i):
        for k in range(D // 16):
            zbuf[i, pl.ds(k * 16, 16)] = jnp.zeros((16,), jnp.float32)

    for q in range(ROW_MAIN // ZROWS):
        pltpu.sync_copy(zbuf, acc_sh.at[pl.ds(s * ROW_MAIN + q * ZROWS, ZROWS)])

    @pl.when(s == NS - 1)
    def _():
        pltpu.sync_copy(zbuf.at[pl.ds(0, ROW_TAIL)],
                        acc_sh.at[pl.ds(NS * ROW_MAIN, ROW_TAIL)])

    plsc.subcore_barrier()

    @pl.loop(0, NCHUNK)
    def _(t0):
        pltpu.sync_copy(src_hbm.at[wid, pl.ds(t0 * CH, CH)], srcc)
        pltpu.sync_copy(rel_hbm.at[wid, pl.ds(t0 * CH, CH)], relc)
        pltpu.sync_copy(dst_hbm.at[wid, pl.ds(t0 * CH, CH)], dstc)
        for tt in range(CH):
            for k in range(WIN // 16):
                sv = srcc[tt, 0, pl.ds(k * 16, 16)]
                rv = relc[tt, 0, pl.ds(k * 16, 16)]
                dv = dstc[tt, 0, pl.ds(k * 16, 16)]
                segv[0, pl.ds(k * 16, 16)] = dv * R + rv
                srv[0, pl.ds(k * 16, 16)] = sv * R + rv
            pltpu.sync_copy(inv_hbm.at[segv.at[0]], invv)
            pltpu.sync_copy(y_hbm.at[srv.at[0]], rows)
            for k in range(WIN // 16):
                iv = invv[pl.ds(k * 16, 16)]
                for jj in range(16):
                    j = k * 16 + jj
                    sc = iv[jj]
                    for cc in range(D // 16):
                        rows[j, pl.ds(cc * 16, 16)] = (
                            rows[j, pl.ds(cc * 16, 16)] * sc)
            pltpu.sync_copy(rows, acc_sh.at[dstc.at[tt, 0]], add=True)

    plsc.subcore_barrier()
    for q in range(ROW_MAIN // ZROWS):
        off = s * ROW_MAIN + q * ZROWS
        pltpu.sync_copy(acc_sh.at[pl.ds(off, ZROWS)], zbuf)
        pltpu.sync_copy(zbuf, out_hbm.at[c, pl.ds(off, ZROWS)])

    @pl.when(s == NS - 1)
    def _():
        pltpu.sync_copy(acc_sh.at[pl.ds(NS * ROW_MAIN, ROW_TAIL)],
                        zbuf.at[pl.ds(0, ROW_TAIL)])
        pltpu.sync_copy(zbuf.at[pl.ds(0, ROW_TAIL)],
                        out_hbm.at[c, pl.ds(NS * ROW_MAIN, ROW_TAIL)])


def _agg_call(src4d, dst4d, rel4d, inv, y):
    return pl.kernel(
        _agg_body,
        out_type=jax.ShapeDtypeStruct((NC, N, D), jnp.float32),
        mesh=_MESH,
        scratch_types=[
            pltpu.VMEM((CH, 1, WIN), jnp.int32),
            pltpu.VMEM((CH, 1, WIN), jnp.int32),
            pltpu.VMEM((CH, 1, WIN), jnp.int32),
            pltpu.VMEM((1, WIN), jnp.int32),
            pltpu.VMEM((1, WIN), jnp.int32),
            pltpu.VMEM((WIN,), jnp.float32),
            pltpu.VMEM((WIN, D), jnp.float32),
            pltpu.VMEM((ZROWS, D), jnp.float32),
            pltpu.VMEM_SHARED((N, D), jnp.float32),
        ],
    )(src4d, dst4d, rel4d, inv, y)


def _inv_kernel(d0_ref, d1_ref, inv_ref):
    inv_ref[...] = 1.0 / (d0_ref[...] + d1_ref[...] + EPS)


def _inv_call(d0, d1):
    return pl.pallas_call(
        _inv_kernel,
        out_shape=jax.ShapeDtypeStruct((SEG // 128, 128), jnp.float32),
    )(d0, d1)


def _mm_kernel(h_ref, wt_ref, ws_ref, bias_ref, y_ref, self_ref):
    h = h_ref[...]
    y_ref[...] = jnp.dot(h, wt_ref[...], preferred_element_type=jnp.float32)
    self_ref[...] = (jnp.dot(h, ws_ref[...], preferred_element_type=jnp.float32)
                     + bias_ref[...])


def _mm_call(h, wt, ws, bias):
    return pl.pallas_call(
        _mm_kernel,
        grid=(_NBLK,),
        in_specs=[
            pl.BlockSpec((_BROW, D), lambda i: (i, 0)),
            pl.BlockSpec((D, R * D), lambda i: (0, 0)),
            pl.BlockSpec((D, D), lambda i: (0, 0)),
            pl.BlockSpec((1, D), lambda i: (0, 0)),
        ],
        out_specs=[
            pl.BlockSpec((_BROW, R * D), lambda i: (i, 0)),
            pl.BlockSpec((_BROW, D), lambda i: (i, 0)),
        ],
        out_shape=[
            jax.ShapeDtypeStruct((N, R * D), jnp.float32),
            jax.ShapeDtypeStruct((N, D), jnp.float32),
        ],
    )(h, wt, ws, bias)


def _comb_kernel(a0_ref, a1_ref, self_ref, o_ref):
    o_ref[...] = jnp.maximum(a0_ref[...] + a1_ref[...] + self_ref[...], 0.0)


def _comb_call(a0, a1, selfp):
    return pl.pallas_call(
        _comb_kernel,
        grid=(_NBLK,),
        in_specs=[pl.BlockSpec((_BROW, D), lambda i: (i, 0))] * 3,
        out_specs=pl.BlockSpec((_BROW, D), lambda i: (i, 0)),
        out_shape=jax.ShapeDtypeStruct((N, D), jnp.float32),
    )(a0, a1, selfp)


def _gsum_kernel(h_ref, g_ref):
    i = pl.program_id(0)
    psum = jnp.sum(h_ref[...], axis=0, keepdims=True)

    @pl.when(i == 0)
    def _():
        g_ref[...] = psum

    @pl.when(i > 0)
    def _():
        g_ref[...] += psum


def _gsum_call(h):
    return pl.pallas_call(
        _gsum_kernel,
        grid=(_NBLK,),
        in_specs=[pl.BlockSpec((_BROW, D), lambda i: (i, 0))],
        out_specs=pl.BlockSpec((1, D), lambda i: (0, 0)),
        out_shape=jax.ShapeDtypeStruct((1, D), jnp.float32),
    )(h)


def kernel(x, edge_index, edge_type, W1, b1, W1s, b1s, W2, b2, W2s, b2s):
    src4d = edge_index[0].reshape(NW, NWIN, 1, WIN)
    dst4d = edge_index[1].reshape(NW, NWIN, 1, WIN)
    rel4d = edge_type.reshape(NW, NWIN, 1, WIN)

    den2 = _den_call(dst4d, rel4d).reshape(NC, SEG)
    inv = _inv_call(den2[0].reshape(SEG // 128, 128),
                    den2[1].reshape(SEG // 128, 128)).reshape(SEG)

    wt1 = W1.reshape(R, D, D).transpose(1, 0, 2).reshape(D, R * D)
    wt2 = W2.reshape(R, D, D).transpose(1, 0, 2).reshape(D, R * D)
    wts = jnp.stack([wt1, wt2])
    wss = jnp.stack([W1s, W2s])
    biases = jnp.stack([(b1 + b1s).reshape(1, D), (b2 + b2s).reshape(1, D)])

    def body(h, xs):
        wt, ws, bias = xs
        y, selfp = _mm_call(h, wt, ws, bias)
        acc = _agg_call(src4d, dst4d, rel4d, inv, y.reshape(SEG, D))
        return _comb_call(acc[0], acc[1], selfp), None

    h2, _ = lax.scan(body, x, (wts, wss, biases))
    graph = _gsum_call(h2)
    return (graph, h2)


# speedup vs baseline: 4.6267x; 1.1784x over previous
import jax
import jax.numpy as jnp
from jax import lax
from jax.experimental import pallas as pl
from jax.experimental.pallas import tpu as pltpu
from jax.experimental.pallas import tpu_sc as plsc

N = 10000
E = 320000
R = 16
D = 128
EPS = 1e-10

NC = 2
NS = 16
NW = NC * NS
EPW = E // NW
WIN = 80
NWIN = EPW // WIN
SEG = N * R
SEG_PER_SUB = SEG // NS
ROW_MAIN = 624
ROW_TAIL = N - NS * ROW_MAIN

CH = 5
NCHUNK = NWIN // CH
ZROWS = 48
DZ = 2000

_MESH = plsc.VectorSubcoreMesh(core_axis_name="c", subcore_axis_name="s")

_NBLK = 25
_BROW = N // _NBLK


def _den_body(src_hbm, dst_hbm, rel_hbm, den2_hbm, seg_hbm, sr_hbm,
              srcc, dstc, relc, segc, srvc, onesv, zbuf, den_sh):
    c = lax.axis_index("c")
    s = lax.axis_index("s")
    wid = c * NS + s

    @pl.loop(0, DZ // 16)
    def _(i):
        zbuf[pl.ds(i * 16, 16)] = jnp.zeros((16,), jnp.float32)

    for q in range(SEG_PER_SUB // DZ):
        pltpu.sync_copy(zbuf, den_sh.at[pl.ds(s * SEG_PER_SUB + q * DZ, DZ)])

    for k in range(WIN // 16):
        onesv[pl.ds(k * 16, 16)] = jnp.ones((16,), jnp.float32)

    plsc.subcore_barrier()

    @pl.loop(0, NCHUNK)
    def _(t0):
        pltpu.sync_copy(src_hbm.at[wid, pl.ds(t0 * CH, CH)], srcc)
        pltpu.sync_copy(dst_hbm.at[wid, pl.ds(t0 * CH, CH)], dstc)
        pltpu.sync_copy(rel_hbm.at[wid, pl.ds(t0 * CH, CH)], relc)
        for tt in range(CH):
            for k in range(WIN // 16):
                sv = srcc[tt, 0, pl.ds(k * 16, 16)]
                dv = dstc[tt, 0, pl.ds(k * 16, 16)]
                rv = relc[tt, 0, pl.ds(k * 16, 16)]
                segc[tt, 0, pl.ds(k * 16, 16)] = dv * R + rv
                srvc[tt, 0, pl.ds(k * 16, 16)] = sv * R + rv
            pltpu.sync_copy(onesv, den_sh.at[segc.at[tt, 0]], add=True)
        pltpu.sync_copy(segc, seg_hbm.at[wid, pl.ds(t0 * CH, CH)])
        pltpu.sync_copy(srvc, sr_hbm.at[wid, pl.ds(t0 * CH, CH)])

    plsc.subcore_barrier()
    for q in range(SEG_PER_SUB // DZ):
        pltpu.sync_copy(den_sh.at[pl.ds(s * SEG_PER_SUB + q * DZ, DZ)], zbuf)
        pltpu.sync_copy(zbuf,
                        den2_hbm.at[pl.ds(c * SEG + s * SEG_PER_SUB + q * DZ,
                                          DZ)])


def _den_call(src4d, dst4d, rel4d):
    return pl.kernel(
        _den_body,
        out_type=[
            jax.ShapeDtypeStruct((NC * SEG,), jnp.float32),
            jax.ShapeDtypeStruct((NW, NWIN, 1, WIN), jnp.int32),
            jax.ShapeDtypeStruct((NW, NWIN, 1, WIN), jnp.int32),
        ],
        mesh=_MESH,
        scratch_types=[
            pltpu.VMEM((CH, 1, WIN), jnp.int32),
            pltpu.VMEM((CH, 1, WIN), jnp.int32),
            pltpu.VMEM((CH, 1, WIN), jnp.int32),
            pltpu.VMEM((CH, 1, WIN), jnp.int32),
            pltpu.VMEM((CH, 1, WIN), jnp.int32),
            pltpu.VMEM((WIN,), jnp.float32),
            pltpu.VMEM((DZ,), jnp.float32),
            pltpu.VMEM_SHARED((SEG,), jnp.float32),
        ],
    )(src4d, dst4d, rel4d)


def _scale_body(seg_hbm, inv_hbm, scale_hbm, segc, scc):
    c = lax.axis_index("c")
    s = lax.axis_index("s")
    wid = c * NS + s

    @pl.loop(0, NCHUNK)
    def _(t0):
        pltpu.sync_copy(seg_hbm.at[wid, pl.ds(t0 * CH, CH)], segc)
        for tt in range(CH):
            pltpu.sync_copy(inv_hbm.at[segc.at[tt, 0]], scc.at[tt, 0])
        pltpu.sync_copy(scc, scale_hbm.at[wid, pl.ds(t0 * CH, CH)])


def _scale_call(seg4d, inv):
    return pl.kernel(
        _scale_body,
        out_type=jax.ShapeDtypeStruct((NW, NWIN, 1, WIN), jnp.float32),
        mesh=_MESH,
        scratch_types=[
            pltpu.VMEM((CH, 1, WIN), jnp.int32),
            pltpu.VMEM((CH, 1, WIN), jnp.float32),
        ],
    )(seg4d, inv)


def _make_agg_body():
    def body(sr_hbm, dst_hbm, scale_hbm, y_hbm, out_hbm,
             srv4, dstv4, scv4, rows2, zbuf, acc_sh,
             semi0, semi1, semi2, semi3, semg0, semg1, sems0, sems1):
        c = lax.axis_index("c")
        s = lax.axis_index("s")
        wid = c * NS + s
        semi = (semi0, semi1, semi2, semi3)
        semg = (semg0, semg1)
        sems = (sems0, sems1)

        @pl.loop(0, ZROWS)
        def _(i):
            for k in range(D // 16):
                zbuf[i, pl.ds(k * 16, 16)] = jnp.zeros((16,), jnp.float32)

        for q in range(ROW_MAIN // ZROWS):
            pltpu.sync_copy(zbuf,
                            acc_sh.at[pl.ds(s * ROW_MAIN + q * ZROWS, ZROWS)])

        @pl.when(s == NS - 1)
        def _():
            pltpu.sync_copy(zbuf.at[pl.ds(0, ROW_TAIL)],
                            acc_sh.at[pl.ds(NS * ROW_MAIN, ROW_TAIL)])

        plsc.subcore_barrier()

        def idx_descs(w, sl):
            return (
                pltpu.make_async_copy(sr_hbm.at[wid, w], srv4.at[sl], semi[sl]),
                pltpu.make_async_copy(dst_hbm.at[wid, w], dstv4.at[sl],
                                      semi[sl]),
                pltpu.make_async_copy(scale_hbm.at[wid, w], scv4.at[sl],
                                      semi[sl]),
            )

        def fetch_idx(w, sl):
            for d in idx_descs(w, sl):
                d.start()

        def wait_idx(w, sl):
            for d in idx_descs(w, sl):
                d.wait()

        def gather_desc(sl4, b):
            return pltpu.make_async_copy(y_hbm.at[srv4.at[sl4, 0]],
                                         rows2.at[b], semg[b])

        def scatter_desc(b, sl4):
            return pltpu.make_async_copy(rows2.at[b],
                                         acc_sh.at[dstv4.at[sl4, 0]], sems[b])

        def proc(w, j):
            b = j % 2
            gather_desc(j, b).wait()
            for k in range(WIN // 16):
                iv = scv4[j, 0, pl.ds(k * 16, 16)]
                for jj in range(16):
                    row = k * 16 + jj
                    sc = iv[jj]
                    for cc in range(D // 16):
                        rows2[b, row, pl.ds(cc * 16, 16)] = (
                            rows2[b, row, pl.ds(cc * 16, 16)] * sc)
            scatter_desc(b, j).start(add=True)

            @pl.when(w >= 1)
            def _():
                scatter_desc((b + 1) % 2, (j - 1) % 4).wait()

            @pl.when(w + 1 <= NWIN - 1)
            def _():
                wait_idx(w + 1, (j + 1) % 4)
                gather_desc((j + 1) % 4, (b + 1) % 2).start()

            @pl.when(w + 2 <= NWIN - 1)
            def _():
                fetch_idx(w + 2, (j + 2) % 4)

        fetch_idx(0, 0)
        fetch_idx(1, 1)
        wait_idx(0, 0)
        gather_desc(0, 0).start()

        @pl.loop(0, (NWIN - 1) // 4)
        def _(m):
            for j in range(4):
                proc(m * 4 + j, j)

        proc(NWIN - 1, (NWIN - 1) % 4)

        scatter_desc((NWIN - 1) % 2, (NWIN - 1) % 4).wait()

        plsc.subcore_barrier()
        for q in range(ROW_MAIN // ZROWS):
            off = s * ROW_MAIN + q * ZROWS
            pltpu.sync_copy(acc_sh.at[pl.ds(off, ZROWS)], zbuf)
            pltpu.sync_copy(zbuf, out_hbm.at[c, pl.ds(off, ZROWS)])

        @pl.when(s == NS - 1)
        def _():
            pltpu.sync_copy(acc_sh.at[pl.ds(NS * ROW_MAIN, ROW_TAIL)],
                            zbuf.at[pl.ds(0, ROW_TAIL)])
            pltpu.sync_copy(zbuf.at[pl.ds(0, ROW_TAIL)],
                            out_hbm.at[c, pl.ds(NS * ROW_MAIN, ROW_TAIL)])

    return body


def _agg_call(sr4d, dst4d, scale4d, y):
    return pl.kernel(
        _make_agg_body(),
        out_type=jax.ShapeDtypeStruct((NC, N, D), jnp.float32),
        mesh=_MESH,
        scratch_types=[
            pltpu.VMEM((4, 1, WIN), jnp.int32),
            pltpu.VMEM((4, 1, WIN), jnp.int32),
            pltpu.VMEM((4, 1, WIN), jnp.float32),
            pltpu.VMEM((2, WIN, D), jnp.float32),
            pltpu.VMEM((ZROWS, D), jnp.float32),
            pltpu.VMEM_SHARED((N, D), jnp.float32),
            pltpu.SemaphoreType.DMA,
            pltpu.SemaphoreType.DMA,
            pltpu.SemaphoreType.DMA,
            pltpu.SemaphoreType.DMA,
            pltpu.SemaphoreType.DMA,
            pltpu.SemaphoreType.DMA,
            pltpu.SemaphoreType.DMA,
            pltpu.SemaphoreType.DMA,
        ],
    )(sr4d, dst4d, scale4d, y)


def _inv_kernel(d0_ref, d1_ref, inv_ref):
    inv_ref[...] = 1.0 / (d0_ref[...] + d1_ref[...] + EPS)


def _inv_call(d0, d1):
    return pl.pallas_call(
        _inv_kernel,
        out_shape=jax.ShapeDtypeStruct((SEG // 128, 128), jnp.float32),
    )(d0, d1)


def _mm_kernel(h_ref, wt_ref, ws_ref, bias_ref, y_ref, self_ref):
    h = h_ref[...]
    y_ref[...] = jnp.dot(h, wt_ref[...], preferred_element_type=jnp.float32)
    self_ref[...] = (jnp.dot(h, ws_ref[...], preferred_element_type=jnp.float32)
                     + bias_ref[...])


def _mm_call(h, wt, ws, bias):
    return pl.pallas_call(
        _mm_kernel,
        grid=(_NBLK,),
        in_specs=[
            pl.BlockSpec((_BROW, D), lambda i: (i, 0)),
            pl.BlockSpec((D, R * D), lambda i: (0, 0)),
            pl.BlockSpec((D, D), lambda i: (0, 0)),
            pl.BlockSpec((1, D), lambda i: (0, 0)),
        ],
        out_specs=[
            pl.BlockSpec((_BROW, R * D), lambda i: (i, 0)),
            pl.BlockSpec((_BROW, D), lambda i: (i, 0)),
        ],
        out_shape=[
            jax.ShapeDtypeStruct((N, R * D), jnp.float32),
            jax.ShapeDtypeStruct((N, D), jnp.float32),
        ],
    )(h, wt, ws, bias)


def _comb_kernel(a0_ref, a1_ref, self_ref, o_ref):
    o_ref[...] = jnp.maximum(a0_ref[...] + a1_ref[...] + self_ref[...], 0.0)


def _comb_call(a0, a1, selfp):
    return pl.pallas_call(
        _comb_kernel,
        grid=(_NBLK,),
        in_specs=[pl.BlockSpec((_BROW, D), lambda i: (i, 0))] * 3,
        out_specs=pl.BlockSpec((_BROW, D), lambda i: (i, 0)),
        out_shape=jax.ShapeDtypeStruct((N, D), jnp.float32),
    )(a0, a1, selfp)


def _gsum_kernel(h_ref, g_ref):
    i = pl.program_id(0)
    psum = jnp.sum(h_ref[...], axis=0, keepdims=True)

    @pl.when(i == 0)
    def _():
        g_ref[...] = psum

    @pl.when(i > 0)
    def _():
        g_ref[...] += psum


def _gsum_call(h):
    return pl.pallas_call(
        _gsum_kernel,
        grid=(_NBLK,),
        in_specs=[pl.BlockSpec((_BROW, D), lambda i: (i, 0))],
        out_specs=pl.BlockSpec((1, D), lambda i: (0, 0)),
        out_shape=jax.ShapeDtypeStruct((1, D), jnp.float32),
    )(h)


def kernel(x, edge_index, edge_type, W1, b1, W1s, b1s, W2, b2, W2s, b2s):
    src4d = edge_index[0].reshape(NW, NWIN, 1, WIN)
    dst4d = edge_index[1].reshape(NW, NWIN, 1, WIN)
    rel4d = edge_type.reshape(NW, NWIN, 1, WIN)

    den2flat, seg4d, sr4d = _den_call(src4d, dst4d, rel4d)
    den2 = den2flat.reshape(NC, SEG)
    inv = _inv_call(den2[0].reshape(SEG // 128, 128),
                    den2[1].reshape(SEG // 128, 128)).reshape(SEG)
    scale4d = _scale_call(seg4d, inv)

    wt1 = W1.reshape(R, D, D).transpose(1, 0, 2).reshape(D, R * D)
    wt2 = W2.reshape(R, D, D).transpose(1, 0, 2).reshape(D, R * D)
    wts = jnp.stack([wt1, wt2])
    wss = jnp.stack([W1s, W2s])
    biases = jnp.stack([(b1 + b1s).reshape(1, D), (b2 + b2s).reshape(1, D)])

    def body(h, xs):
        wt, ws, bias = xs
        y, selfp = _mm_call(h, wt, ws, bias)
        acc = _agg_call(sr4d, dst4d, scale4d, y.reshape(SEG, D))
        return _comb_call(acc[0], acc[1], selfp), None

    h2, _ = lax.scan(body, x, (wts, wss, biases))
    graph = _gsum_call(h2)
    return (graph, h2)


# speedup vs baseline: 5.0775x; 1.0974x over previous
import jax
import jax.numpy as jnp
from jax import lax
from jax.experimental import pallas as pl
from jax.experimental.pallas import tpu as pltpu
from jax.experimental.pallas import tpu_sc as plsc

N = 10000
E = 320000
R = 16
D = 128
EPS = 1e-10

NC = 2
NS = 16
NW = NC * NS
EPW = E // NW
WIN = 80
NWIN = EPW // WIN
SEG = N * R
SEG_PER_SUB = SEG // NS
ROW_MAIN = 624
ROW_TAIL = N - NS * ROW_MAIN

CH = 5
NCHUNK = NWIN // CH
ZROWS = 48
DZ = 2000

_MESH = plsc.VectorSubcoreMesh(core_axis_name="c", subcore_axis_name="s")

_NBLK = 25
_BROW = N // _NBLK


def _den_body(src_hbm, dst_hbm, rel_hbm, den2_hbm, seg_hbm, sr_hbm,
              srcc, dstc, relc, segc, srvc, onesv, zbuf, den_sh):
    c = lax.axis_index("c")
    s = lax.axis_index("s")
    wid = c * NS + s

    @pl.loop(0, DZ // 16)
    def _(i):
        zbuf[pl.ds(i * 16, 16)] = jnp.zeros((16,), jnp.float32)

    for q in range(SEG_PER_SUB // DZ):
        pltpu.sync_copy(zbuf, den_sh.at[pl.ds(s * SEG_PER_SUB + q * DZ, DZ)])

    for k in range(WIN // 16):
        onesv[pl.ds(k * 16, 16)] = jnp.ones((16,), jnp.float32)

    plsc.subcore_barrier()

    @pl.loop(0, NCHUNK)
    def _(t0):
        pltpu.sync_copy(src_hbm.at[wid, pl.ds(t0 * CH, CH)], srcc)
        pltpu.sync_copy(dst_hbm.at[wid, pl.ds(t0 * CH, CH)], dstc)
        pltpu.sync_copy(rel_hbm.at[wid, pl.ds(t0 * CH, CH)], relc)
        for tt in range(CH):
            for k in range(WIN // 16):
                sv = srcc[tt, 0, pl.ds(k * 16, 16)]
                dv = dstc[tt, 0, pl.ds(k * 16, 16)]
                rv = relc[tt, 0, pl.ds(k * 16, 16)]
                segc[tt, 0, pl.ds(k * 16, 16)] = dv * R + rv
                srvc[tt, 0, pl.ds(k * 16, 16)] = sv * R + rv
            pltpu.sync_copy(onesv, den_sh.at[segc.at[tt, 0]], add=True)
        pltpu.sync_copy(segc, seg_hbm.at[wid, pl.ds(t0 * CH, CH)])
        pltpu.sync_copy(srvc, sr_hbm.at[wid, pl.ds(t0 * CH, CH)])

    plsc.subcore_barrier()
    for q in range(SEG_PER_SUB // DZ):
        pltpu.sync_copy(den_sh.at[pl.ds(s * SEG_PER_SUB + q * DZ, DZ)], zbuf)
        pltpu.sync_copy(zbuf,
                        den2_hbm.at[pl.ds(c * SEG + s * SEG_PER_SUB + q * DZ,
                                          DZ)])


def _den_call(src4d, dst4d, rel4d):
    return pl.kernel(
        _den_body,
        out_type=[
            jax.ShapeDtypeStruct((NC * SEG,), jnp.float32),
            jax.ShapeDtypeStruct((NW, NWIN, 1, WIN), jnp.int32),
            jax.ShapeDtypeStruct((NW, NWIN, 1, WIN), jnp.int32),
        ],
        mesh=_MESH,
        scratch_types=[
            pltpu.VMEM((CH, 1, WIN), jnp.int32),
            pltpu.VMEM((CH, 1, WIN), jnp.int32),
            pltpu.VMEM((CH, 1, WIN), jnp.int32),
            pltpu.VMEM((CH, 1, WIN), jnp.int32),
            pltpu.VMEM((CH, 1, WIN), jnp.int32),
            pltpu.VMEM((WIN,), jnp.float32),
            pltpu.VMEM((DZ,), jnp.float32),
            pltpu.VMEM_SHARED((SEG,), jnp.float32),
        ],
    )(src4d, dst4d, rel4d)


def _scale_body(seg_hbm, inv_hbm, scale_hbm, segc, scc):
    c = lax.axis_index("c")
    s = lax.axis_index("s")
    wid = c * NS + s

    @pl.loop(0, NCHUNK)
    def _(t0):
        pltpu.sync_copy(seg_hbm.at[wid, pl.ds(t0 * CH, CH)], segc)
        for tt in range(CH):
            pltpu.sync_copy(inv_hbm.at[segc.at[tt, 0]], scc.at[tt, 0])
        pltpu.sync_copy(scc, scale_hbm.at[wid, pl.ds(t0 * CH, CH)])


def _scale_call(seg4d, inv):
    return pl.kernel(
        _scale_body,
        out_type=jax.ShapeDtypeStruct((NW, NWIN, 1, WIN), jnp.float32),
        mesh=_MESH,
        scratch_types=[
            pltpu.VMEM((CH, 1, WIN), jnp.int32),
            pltpu.VMEM((CH, 1, WIN), jnp.float32),
        ],
    )(seg4d, inv)


def _make_agg_body():
    def body(sr_hbm, dst_hbm, scale_hbm, y_hbm, out_hbm,
             srv4, dstv4, scv4, rows2, zbuf, acc_sh,
             semi0, semi1, semi2, semi3, semg0, semg1, sems0, sems1):
        c = lax.axis_index("c")
        s = lax.axis_index("s")
        wid = c * NS + s
        semi = (semi0, semi1, semi2, semi3)
        semg = (semg0, semg1)
        sems = (sems0, sems1)

        @pl.loop(0, ZROWS)
        def _(i):
            for k in range(D // 16):
                zbuf[i, pl.ds(k * 16, 16)] = jnp.zeros((16,), jnp.float32)

        for q in range(ROW_MAIN // ZROWS):
            pltpu.sync_copy(zbuf,
                            acc_sh.at[pl.ds(s * ROW_MAIN + q * ZROWS, ZROWS)])

        @pl.when(s == NS - 1)
        def _():
            pltpu.sync_copy(zbuf.at[pl.ds(0, ROW_TAIL)],
                            acc_sh.at[pl.ds(NS * ROW_MAIN, ROW_TAIL)])

        plsc.subcore_barrier()

        def idx_descs(w, sl):
            return (
                pltpu.make_async_copy(sr_hbm.at[wid, w], srv4.at[sl], semi[sl]),
                pltpu.make_async_copy(dst_hbm.at[wid, w], dstv4.at[sl],
                                      semi[sl]),
                pltpu.make_async_copy(scale_hbm.at[wid, w], scv4.at[sl],
                                      semi[sl]),
            )

        def fetch_idx(w, sl):
            for d in idx_descs(w, sl):
                d.start()

        def wait_idx(w, sl):
            for d in idx_descs(w, sl):
                d.wait()

        def gather_desc(sl4, b):
            return pltpu.make_async_copy(y_hbm.at[srv4.at[sl4, 0]],
                                         rows2.at[b], semg[b])

        def scatter_desc(b, sl4):
            return pltpu.make_async_copy(rows2.at[b],
                                         acc_sh.at[dstv4.at[sl4, 0]], sems[b])

        def proc(w, j):
            b = j % 2
            gather_desc(j, b).wait()

            @pl.when(w >= 1)
            def _():
                scatter_desc((b + 1) % 2, (j - 1) % 4).wait()

            @pl.when(w + 1 <= NWIN - 1)
            def _():
                wait_idx(w + 1, (j + 1) % 4)
                gather_desc((j + 1) % 4, (b + 1) % 2).start()

            for k in range(WIN // 16):
                iv = scv4[j, 0, pl.ds(k * 16, 16)]
                for jj in range(16):
                    row = k * 16 + jj
                    sc = iv[jj]
                    for cc in range(D // 16):
                        rows2[b, row, pl.ds(cc * 16, 16)] = (
                            rows2[b, row, pl.ds(cc * 16, 16)] * sc)
            scatter_desc(b, j).start(add=True)

            @pl.when(w + 2 <= NWIN - 1)
            def _():
                fetch_idx(w + 2, (j + 2) % 4)

        fetch_idx(0, 0)
        fetch_idx(1, 1)
        wait_idx(0, 0)
        gather_desc(0, 0).start()

        @pl.loop(0, (NWIN - 1) // 4)
        def _(m):
            for j in range(4):
                proc(m * 4 + j, j)

        proc(NWIN - 1, (NWIN - 1) % 4)

        scatter_desc((NWIN - 1) % 2, (NWIN - 1) % 4).wait()

        plsc.subcore_barrier()
        for q in range(ROW_MAIN // ZROWS):
            off = s * ROW_MAIN + q * ZROWS
            pltpu.sync_copy(acc_sh.at[pl.ds(off, ZROWS)], zbuf)
            pltpu.sync_copy(zbuf, out_hbm.at[c, pl.ds(off, ZROWS)])

        @pl.when(s == NS - 1)
        def _():
            pltpu.sync_copy(acc_sh.at[pl.ds(NS * ROW_MAIN, ROW_TAIL)],
                            zbuf.at[pl.ds(0, ROW_TAIL)])
            pltpu.sync_copy(zbuf.at[pl.ds(0, ROW_TAIL)],
                            out_hbm.at[c, pl.ds(NS * ROW_MAIN, ROW_TAIL)])

    return body


def _agg_call(sr4d, dst4d, scale4d, y):
    return pl.kernel(
        _make_agg_body(),
        out_type=jax.ShapeDtypeStruct((NC, N, D), jnp.float32),
        mesh=_MESH,
        scratch_types=[
            pltpu.VMEM((4, 1, WIN), jnp.int32),
            pltpu.VMEM((4, 1, WIN), jnp.int32),
            pltpu.VMEM((4, 1, WIN), jnp.float32),
            pltpu.VMEM((2, WIN, D), jnp.float32),
            pltpu.VMEM((ZROWS, D), jnp.float32),
            pltpu.VMEM_SHARED((N, D), jnp.float32),
            pltpu.SemaphoreType.DMA,
            pltpu.SemaphoreType.DMA,
            pltpu.SemaphoreType.DMA,
            pltpu.SemaphoreType.DMA,
            pltpu.SemaphoreType.DMA,
            pltpu.SemaphoreType.DMA,
            pltpu.SemaphoreType.DMA,
            pltpu.SemaphoreType.DMA,
        ],
    )(sr4d, dst4d, scale4d, y)


def _inv_kernel(d0_ref, d1_ref, inv_ref):
    inv_ref[...] = 1.0 / (d0_ref[...] + d1_ref[...] + EPS)


def _inv_call(d0, d1):
    return pl.pallas_call(
        _inv_kernel,
        out_shape=jax.ShapeDtypeStruct((SEG // 128, 128), jnp.float32),
    )(d0, d1)


def _mm_kernel(h_ref, wt_ref, ws_ref, bias_ref, y_ref, self_ref):
    h = h_ref[...]
    y_ref[...] = jnp.dot(h, wt_ref[...], preferred_element_type=jnp.float32)
    self_ref[...] = (jnp.dot(h, ws_ref[...], preferred_element_type=jnp.float32)
                     + bias_ref[...])


def _mm_call(h, wt, ws, bias):
    return pl.pallas_call(
        _mm_kernel,
        grid=(_NBLK,),
        in_specs=[
            pl.BlockSpec((_BROW, D), lambda i: (i, 0)),
            pl.BlockSpec((D, R * D), lambda i: (0, 0)),
            pl.BlockSpec((D, D), lambda i: (0, 0)),
            pl.BlockSpec((1, D), lambda i: (0, 0)),
        ],
        out_specs=[
            pl.BlockSpec((_BROW, R * D), lambda i: (i, 0)),
            pl.BlockSpec((_BROW, D), lambda i: (i, 0)),
        ],
        out_shape=[
            jax.ShapeDtypeStruct((N, R * D), jnp.float32),
            jax.ShapeDtypeStruct((N, D), jnp.float32),
        ],
    )(h, wt, ws, bias)


def _comb_kernel(a0_ref, a1_ref, self_ref, o_ref):
    o_ref[...] = jnp.maximum(a0_ref[...] + a1_ref[...] + self_ref[...], 0.0)


def _comb_call(a0, a1, selfp):
    return pl.pallas_call(
        _comb_kernel,
        grid=(_NBLK,),
        in_specs=[pl.BlockSpec((_BROW, D), lambda i: (i, 0))] * 3,
        out_specs=pl.BlockSpec((_BROW, D), lambda i: (i, 0)),
        out_shape=jax.ShapeDtypeStruct((N, D), jnp.float32),
    )(a0, a1, selfp)


def _gsum_kernel(h_ref, g_ref):
    i = pl.program_id(0)
    psum = jnp.sum(h_ref[...], axis=0, keepdims=True)

    @pl.when(i == 0)
    def _():
        g_ref[...] = psum

    @pl.when(i > 0)
    def _():
        g_ref[...] += psum


def _gsum_call(h):
    return pl.pallas_call(
        _gsum_kernel,
        grid=(_NBLK,),
        in_specs=[pl.BlockSpec((_BROW, D), lambda i: (i, 0))],
        out_specs=pl.BlockSpec((1, D), lambda i: (0, 0)),
        out_shape=jax.ShapeDtypeStruct((1, D), jnp.float32),
    )(h)


def kernel(x, edge_index, edge_type, W1, b1, W1s, b1s, W2, b2, W2s, b2s):
    src4d = edge_index[0].reshape(NW, NWIN, 1, WIN)
    dst4d = edge_index[1].reshape(NW, NWIN, 1, WIN)
    rel4d = edge_type.reshape(NW, NWIN, 1, WIN)

    den2flat, seg4d, sr4d = _den_call(src4d, dst4d, rel4d)
    den2 = den2flat.reshape(NC, SEG)
    inv = _inv_call(den2[0].reshape(SEG // 128, 128),
                    den2[1].reshape(SEG // 128, 128)).reshape(SEG)
    scale4d = _scale_call(seg4d, inv)

    wt1 = W1.reshape(R, D, D).transpose(1, 0, 2).reshape(D, R * D)
    wt2 = W2.reshape(R, D, D).transpose(1, 0, 2).reshape(D, R * D)
    wts = jnp.stack([wt1, wt2])
    wss = jnp.stack([W1s, W2s])
    biases = jnp.stack([(b1 + b1s).reshape(1, D), (b2 + b2s).reshape(1, D)])

    def body(h, xs):
        wt, ws, bias = xs
        y, selfp = _mm_call(h, wt, ws, bias)
        acc = _agg_call(sr4d, dst4d, scale4d, y.reshape(SEG, D))
        return _comb_call(acc[0], acc[1], selfp), None

    h2, _ = lax.scan(body, x, (wts, wss, biases))
    graph = _gsum_call(h2)
    return (graph, h2)


# speedup vs baseline: 5.7834x; 1.1390x over previous
import jax
import jax.numpy as jnp
from jax import lax
from jax.experimental import pallas as pl
from jax.experimental.pallas import tpu as pltpu
from jax.experimental.pallas import tpu_sc as plsc

N = 10000
E = 320000
R = 16
D = 128
EPS = 1e-10

NC = 2
NS = 16
NW = NC * NS
EPW = E // NW
WIN = 80
NWIN = EPW // WIN
SEG = N * R
SEG_PER_SUB = SEG // NS
ROW_MAIN = 624
ROW_TAIL = N - NS * ROW_MAIN

CH = 5
NCHUNK = NWIN // CH
ZROWS = 16
DZ = 1000

_MESH = plsc.VectorSubcoreMesh(core_axis_name="c", subcore_axis_name="s")

_NBLK = 25
_BROW = N // _NBLK


def _den_body(src_hbm, dst_hbm, rel_hbm, den2_hbm, seg_hbm, sr_hbm,
              srcc, dstc, relc, onesv, zbuf, den_sh):
    c = lax.axis_index("c")
    s = lax.axis_index("s")
    wid = c * NS + s

    @pl.loop(0, DZ // 16)
    def _(i):
        zbuf[pl.ds(i * 16, 16)] = jnp.zeros((16,), jnp.float32)

    for q in range(SEG_PER_SUB // DZ):
        pltpu.sync_copy(zbuf, den_sh.at[pl.ds(s * SEG_PER_SUB + q * DZ, DZ)])

    for k in range(WIN // 16):
        onesv[pl.ds(k * 16, 16)] = jnp.ones((16,), jnp.float32)

    plsc.subcore_barrier()

    @pl.loop(0, NCHUNK)
    def _(t0):
        pltpu.sync_copy(src_hbm.at[wid, pl.ds(t0 * CH, CH)], srcc)
        pltpu.sync_copy(dst_hbm.at[wid, pl.ds(t0 * CH, CH)], dstc)
        pltpu.sync_copy(rel_hbm.at[wid, pl.ds(t0 * CH, CH)], relc)
        for tt in range(CH):
            for k in range(WIN // 16):
                sv = srcc[tt, 0, pl.ds(k * 16, 16)]
                dv = dstc[tt, 0, pl.ds(k * 16, 16)]
                rv = relc[tt, 0, pl.ds(k * 16, 16)]
                dstc[tt, 0, pl.ds(k * 16, 16)] = dv * R + rv
                srcc[tt, 0, pl.ds(k * 16, 16)] = sv * R + rv
            pltpu.sync_copy(onesv, den_sh.at[dstc.at[tt, 0]], add=True)
        pltpu.sync_copy(dstc, seg_hbm.at[wid, pl.ds(t0 * CH, CH)])
        pltpu.sync_copy(srcc, sr_hbm.at[wid, pl.ds(t0 * CH, CH)])

    plsc.subcore_barrier()
    for q in range(SEG_PER_SUB // DZ):
        pltpu.sync_copy(den_sh.at[pl.ds(s * SEG_PER_SUB + q * DZ, DZ)], zbuf)
        pltpu.sync_copy(zbuf,
                        den2_hbm.at[pl.ds(c * SEG + s * SEG_PER_SUB + q * DZ,
                                          DZ)])


def _den_call(src4d, dst4d, rel4d):
    return pl.kernel(
        _den_body,
        out_type=[
            jax.ShapeDtypeStruct((NC * SEG,), jnp.float32),
            jax.ShapeDtypeStruct((NW, NWIN, 1, WIN), jnp.int32),
            jax.ShapeDtypeStruct((NW, NWIN, 1, WIN), jnp.int32),
        ],
        mesh=_MESH,
        scratch_types=[
            pltpu.VMEM((CH, 1, WIN), jnp.int32),
            pltpu.VMEM((CH, 1, WIN), jnp.int32),
            pltpu.VMEM((CH, 1, WIN), jnp.int32),
            pltpu.VMEM((WIN,), jnp.float32),
            pltpu.VMEM((DZ,), jnp.float32),
            pltpu.VMEM_SHARED((SEG,), jnp.float32),
        ],
    )(src4d, dst4d, rel4d)


def _scale_body(seg_hbm, inv_hbm, scale_hbm, segc, scc):
    c = lax.axis_index("c")
    s = lax.axis_index("s")
    wid = c * NS + s

    @pl.loop(0, NCHUNK)
    def _(t0):
        pltpu.sync_copy(seg_hbm.at[wid, pl.ds(t0 * CH, CH)], segc)
        for tt in range(CH):
            pltpu.sync_copy(inv_hbm.at[segc.at[tt, 0]], scc.at[tt, 0])
        pltpu.sync_copy(scc, scale_hbm.at[wid, pl.ds(t0 * CH, CH)])


def _scale_call(seg4d, inv):
    return pl.kernel(
        _scale_body,
        out_type=jax.ShapeDtypeStruct((NW, NWIN, 1, WIN), jnp.float32),
        mesh=_MESH,
        scratch_types=[
            pltpu.VMEM((CH, 1, WIN), jnp.int32),
            pltpu.VMEM((CH, 1, WIN), jnp.float32),
        ],
    )(seg4d, inv)


def _make_agg_body():
    def body(sr_hbm, dst_hbm, scale_hbm, y_hbm, out_hbm,
             srv4, dstv4, scv4, rows3, zbuf, acc_sh,
             semi0, semi1, semi2, semi3, semg0, semg1, semg2, sems0, sems1):
        c = lax.axis_index("c")
        s = lax.axis_index("s")
        wid = c * NS + s
        semi = (semi0, semi1, semi2, semi3)
        semg = (semg0, semg1, semg2)
        sems = (sems0, sems1)

        @pl.loop(0, ZROWS)
        def _(i):
            for k in range(D // 16):
                zbuf[i, pl.ds(k * 16, 16)] = jnp.zeros((16,), jnp.float32)

        for q in range(ROW_MAIN // ZROWS):
            pltpu.sync_copy(zbuf,
                            acc_sh.at[pl.ds(s * ROW_MAIN + q * ZROWS, ZROWS)])

        @pl.when(s == NS - 1)
        def _():
            pltpu.sync_copy(zbuf.at[pl.ds(0, ROW_TAIL)],
                            acc_sh.at[pl.ds(NS * ROW_MAIN, ROW_TAIL)])

        plsc.subcore_barrier()

        def idx_descs(w, sl):
            return (
                pltpu.make_async_copy(sr_hbm.at[wid, w], srv4.at[sl], semi[sl]),
                pltpu.make_async_copy(dst_hbm.at[wid, w], dstv4.at[sl],
                                      semi[sl]),
                pltpu.make_async_copy(scale_hbm.at[wid, w], scv4.at[sl],
                                      semi[sl]),
            )

        def fetch_idx(w, sl):
            for d in idx_descs(w, sl):
                d.start()

        def wait_idx(w, sl):
            for d in idx_descs(w, sl):
                d.wait()

        def gather_desc(sl4, b3):
            return pltpu.make_async_copy(y_hbm.at[srv4.at[sl4, 0]],
                                         rows3.at[b3], semg[b3])

        def scatter_desc(b3, sl4, sp):
            return pltpu.make_async_copy(rows3.at[b3],
                                         acc_sh.at[dstv4.at[sl4, 0]], sems[sp])

        def scale_rows(b3, sl4):
            @pl.loop(0, WIN // 16)
            def _(k):
                iv = scv4[sl4, 0, pl.ds(k * 16, 16)]
                for jj in range(16):
                    sc = iv[jj]
                    for cc in range(D // 16):
                        rows3[b3, k * 16 + jj, pl.ds(cc * 16, 16)] = (
                            rows3[b3, k * 16 + jj, pl.ds(cc * 16, 16)] * sc)

        def proc(w, j, static=False):
            b3 = j % 3
            sl4 = j % 4
            sp = j % 2

            def drain_prev():
                scatter_desc((j - 1) % 3, (j - 1) % 4, (j - 1) % 2).wait()

            def launch_next():
                wait_idx(w + 2, (j + 2) % 4)
                gather_desc((j + 2) % 4, (j + 2) % 3).start()

            def prefetch():
                fetch_idx(w + 3, (j + 3) % 4)

            gather_desc(sl4, b3).wait()
            if static:
                if w >= 1:
                    drain_prev()
                if w + 2 <= NWIN - 1:
                    launch_next()
            else:
                pl.when(w >= 1)(drain_prev)
                pl.when(w + 2 <= NWIN - 1)(launch_next)

            scale_rows(b3, sl4)
            scatter_desc(b3, sl4, sp).start(add=True)

            if static:
                if w + 3 <= NWIN - 1:
                    prefetch()
            else:
                pl.when(w + 3 <= NWIN - 1)(prefetch)

        fetch_idx(0, 0)
        fetch_idx(1, 1)
        fetch_idx(2, 2)
        wait_idx(0, 0)
        gather_desc(0, 0).start()
        wait_idx(1, 1)
        gather_desc(1, 1).start()

        @pl.loop(0, 10)
        def _(m):
            for j in range(12):
                proc(m * 12 + j, j)

        for w in range(120, NWIN):
            proc(w, w % 12, static=True)

        scatter_desc((NWIN - 1) % 3, (NWIN - 1) % 4, (NWIN - 1) % 2).wait()

        plsc.subcore_barrier()
        for q in range(ROW_MAIN // ZROWS):
            off = s * ROW_MAIN + q * ZROWS
            pltpu.sync_copy(acc_sh.at[pl.ds(off, ZROWS)], zbuf)
            pltpu.sync_copy(zbuf, out_hbm.at[c, pl.ds(off, ZROWS)])

        @pl.when(s == NS - 1)
        def _():
            pltpu.sync_copy(acc_sh.at[pl.ds(NS * ROW_MAIN, ROW_TAIL)],
                            zbuf.at[pl.ds(0, ROW_TAIL)])
            pltpu.sync_copy(zbuf.at[pl.ds(0, ROW_TAIL)],
                            out_hbm.at[c, pl.ds(NS * ROW_MAIN, ROW_TAIL)])

    return body


def _agg_call(sr4d, dst4d, scale4d, y):
    return pl.kernel(
        _make_agg_body(),
        out_type=jax.ShapeDtypeStruct((NC, N, D), jnp.float32),
        mesh=_MESH,
        scratch_types=[
            pltpu.VMEM((4, 1, WIN), jnp.int32),
            pltpu.VMEM((4, 1, WIN), jnp.int32),
            pltpu.VMEM((4, 1, WIN), jnp.float32),
            pltpu.VMEM((3, WIN, D), jnp.float32),
            pltpu.VMEM((ZROWS, D), jnp.float32),
            pltpu.VMEM_SHARED((N, D), jnp.float32),
            pltpu.SemaphoreType.DMA,
            pltpu.SemaphoreType.DMA,
            pltpu.SemaphoreType.DMA,
            pltpu.SemaphoreType.DMA,
            pltpu.SemaphoreType.DMA,
            pltpu.SemaphoreType.DMA,
            pltpu.SemaphoreType.DMA,
            pltpu.SemaphoreType.DMA,
            pltpu.SemaphoreType.DMA,
        ],
    )(sr4d, dst4d, scale4d, y)


def _inv_kernel(d0_ref, d1_ref, inv_ref):
    inv_ref[...] = 1.0 / (d0_ref[...] + d1_ref[...] + EPS)


def _inv_call(d0, d1):
    return pl.pallas_call(
        _inv_kernel,
        out_shape=jax.ShapeDtypeStruct((SEG // 128, 128), jnp.float32),
    )(d0, d1)


def _mm_kernel(h_ref, wt_ref, ws_ref, bias_ref, y_ref, self_ref):
    h = h_ref[...]
    y_ref[...] = jnp.dot(h, wt_ref[...], preferred_element_type=jnp.float32)
    self_ref[...] = (jnp.dot(h, ws_ref[...], preferred_element_type=jnp.float32)
                     + bias_ref[...])


def _mm_call(h, wt, ws, bias):
    return pl.pallas_call(
        _mm_kernel,
        grid=(_NBLK,),
        in_specs=[
            pl.BlockSpec((_BROW, D), lambda i: (i, 0)),
            pl.BlockSpec((D, R * D), lambda i: (0, 0)),
            pl.BlockSpec((D, D), lambda i: (0, 0)),
            pl.BlockSpec((1, D), lambda i: (0, 0)),
        ],
        out_specs=[
            pl.BlockSpec((_BROW, R * D), lambda i: (i, 0)),
            pl.BlockSpec((_BROW, D), lambda i: (i, 0)),
        ],
        out_shape=[
            jax.ShapeDtypeStruct((N, R * D), jnp.float32),
            jax.ShapeDtypeStruct((N, D), jnp.float32),
        ],
    )(h, wt, ws, bias)


def _comb_kernel(a0_ref, a1_ref, self_ref, o_ref):
    o_ref[...] = jnp.maximum(a0_ref[...] + a1_ref[...] + self_ref[...], 0.0)


def _comb_call(a0, a1, selfp):
    return pl.pallas_call(
        _comb_kernel,
        grid=(_NBLK,),
        in_specs=[pl.BlockSpec((_BROW, D), lambda i: (i, 0))] * 3,
        out_specs=pl.BlockSpec((_BROW, D), lambda i: (i, 0)),
        out_shape=jax.ShapeDtypeStruct((N, D), jnp.float32),
    )(a0, a1, selfp)


def _gsum_kernel(h_ref, g_ref):
    i = pl.program_id(0)
    psum = jnp.sum(h_ref[...], axis=0, keepdims=True)

    @pl.when(i == 0)
    def _():
        g_ref[...] = psum

    @pl.when(i > 0)
    def _():
        g_ref[...] += psum


def _gsum_call(h):
    return pl.pallas_call(
        _gsum_kernel,
        grid=(_NBLK,),
        in_specs=[pl.BlockSpec((_BROW, D), lambda i: (i, 0))],
        out_specs=pl.BlockSpec((1, D), lambda i: (0, 0)),
        out_shape=jax.ShapeDtypeStruct((1, D), jnp.float32),
    )(h)


def kernel(x, edge_index, edge_type, W1, b1, W1s, b1s, W2, b2, W2s, b2s):
    src4d = edge_index[0].reshape(NW, NWIN, 1, WIN)
    dst4d = edge_index[1].reshape(NW, NWIN, 1, WIN)
    rel4d = edge_type.reshape(NW, NWIN, 1, WIN)

    den2flat, seg4d, sr4d = _den_call(src4d, dst4d, rel4d)
    den2 = den2flat.reshape(NC, SEG)
    inv = _inv_call(den2[0].reshape(SEG // 128, 128),
                    den2[1].reshape(SEG // 128, 128)).reshape(SEG)
    scale4d = _scale_call(seg4d, inv)

    wt1 = W1.reshape(R, D, D).transpose(1, 0, 2).reshape(D, R * D)
    wt2 = W2.reshape(R, D, D).transpose(1, 0, 2).reshape(D, R * D)
    wts = jnp.stack([wt1, wt2])
    wss = jnp.stack([W1s, W2s])
    biases = jnp.stack([(b1 + b1s).reshape(1, D), (b2 + b2s).reshape(1, D)])

    def body(h, xs):
        wt, ws, bias = xs
        y, selfp = _mm_call(h, wt, ws, bias)
        acc = _agg_call(sr4d, dst4d, scale4d, y.reshape(SEG, D))
        return _comb_call(acc[0], acc[1], selfp), None

    h2, _ = lax.scan(body, x, (wts, wss, biases))
    graph = _gsum_call(h2)
    return (graph, h2)
